# baseline (speedup 1.0000x reference)
import dataclasses
import functools

import numpy as np

import jax
import jax.numpy as jnp
from jax import lax
from jax.experimental import pallas as pl
from jax.experimental.pallas import tpu as pltpu
from jax.experimental.pallas import tpu_sc as plsc

H = 128
CHUNK = 80
NSUB = 16
LANES = 16

def _edge_linear_body(ea_ref, w_ref, b_ref, out_ref):
    a = ea_ref[...].astype(jnp.bfloat16)
    y = jnp.dot(a, w_ref[...], preferred_element_type=jnp.float32) + b_ref[...]
    out_ref[0] = y[:, :H]
    out_ref[1] = y[:, H:]


def _edge_linear(edge_attr, w_stack, b_stack, lo, span, block_e=2000):
    return pl.pallas_call(
        _edge_linear_body,
        grid=(span // block_e,),
        in_specs=[
            pl.BlockSpec((block_e, H), lambda i: (i + lo // block_e, 0)),
            pl.BlockSpec((H, 2 * H), lambda i: (0, 0)),
            pl.BlockSpec((1, 2 * H), lambda i: (0, 0)),
        ],
        out_specs=pl.BlockSpec((2, block_e, H), lambda i: (0, i, 0)),
        out_shape=jax.ShapeDtypeStruct((2, span, H), jnp.float32),
        compiler_params=pltpu.CompilerParams(
            dimension_semantics=("parallel",)
        ),
    )(edge_attr, w_stack, b_stack)


def _sc_aggregate(g_idx, x, e_stack, lo):
    N = x.shape[0]
    E = g_idx.shape[0] // 2
    span = e_stack.shape[1]
    n_chunks = span // CHUNK
    chunks_per_sub = (n_chunks + NSUB - 1) // NSUB
    rows_per_sub = 624
    extra_rows = N - NSUB * rows_per_sub
    full_zero = rows_per_sub // CHUNK
    rem_zero = rows_per_sub % CHUNK

    mesh = plsc.VectorSubcoreMesh(core_axis_name="c", subcore_axis_name="s")

    @functools.partial(
        pl.kernel,
        out_type=jax.ShapeDtypeStruct((2, N, H), jnp.float32),
        mesh=mesh,
        scratch_types=[
            pltpu.VMEM((CHUNK,), jnp.int32),
            pltpu.VMEM((CHUNK,), jnp.int32),
            pltpu.VMEM((CHUNK,), jnp.int32),
            pltpu.VMEM((CHUNK,), jnp.int32),
            pltpu.VMEM((CHUNK, H), jnp.float32),
            pltpu.VMEM((CHUNK, H), jnp.float32),
            pltpu.VMEM((CHUNK, H), jnp.float32),
            pltpu.VMEM((CHUNK, H), jnp.float32),
            pltpu.VMEM_SHARED((N, H), jnp.float32),
        ] + [pltpu.SemaphoreType.DMA] * 10,
    )
    def k(gi_hbm, x_hbm, e_hbm, out_hbm, gidx0, gidx1, sidx0, sidx1, xg0,
          xg1, e0, e1, acc_sh, sem_gi0, sem_gi1, sem_si0, sem_si1, sem_x0,
          sem_x1, sem_e0, sem_e1, sem_s0, sem_s1):
        c = lax.axis_index("c")
        s = lax.axis_index("s")
        gidx = [gidx0, gidx1]
        sidx = [sidx0, sidx1]
        xg = [xg0, xg1]
        ev = [e0, e1]
        sem_gi = [sem_gi0, sem_gi1]
        sem_si = [sem_si0, sem_si1]
        sem_x = [sem_x0, sem_x1]
        sem_e = [sem_e0, sem_e1]
        sem_s = [sem_s0, sem_s1]

        @pl.loop(0, CHUNK)
        def _(i):
            for j in range(H // LANES):
                xg0[i, pl.ds(j * LANES, LANES)] = jnp.zeros(
                    (LANES,), jnp.float32
                )

        base_rows = s * rows_per_sub

        @pl.loop(0, full_zero)
        def _(t):
            pltpu.sync_copy(
                xg0, acc_sh.at[pl.ds(base_rows + t * CHUNK, CHUNK)]
            )

        pltpu.sync_copy(
            xg0.at[pl.ds(0, rem_zero)],
            acc_sh.at[pl.ds(base_rows + full_zero * CHUNK, rem_zero)],
        )

        @pl.when(s == NSUB - 1)
        def _():
            pltpu.sync_copy(
                xg0.at[pl.ds(0, extra_rows)],
                acc_sh.at[pl.ds(NSUB * rows_per_sub, extra_rows)],
            )

        plsc.subcore_barrier()

        def ci_of(t):
            return t * NSUB + s

        def valid(t):
            return ci_of(t) < n_chunks

        def issue_gidx(t, b):
            pltpu.async_copy(
                gi_hbm.at[pl.ds(c * E + lo + ci_of(t) * CHUNK, CHUNK)],
                gidx[b], sem_gi[b],
            )

        def issue_sidx(t, b):
            pltpu.async_copy(
                gi_hbm.at[pl.ds((1 - c) * E + lo + ci_of(t) * CHUNK, CHUNK)],
                sidx[b], sem_si[b],
            )

        def issue_data(t, b):
            pltpu.async_copy(
                e_hbm.at[c, pl.ds(ci_of(t) * CHUNK, CHUNK)], ev[b],
                sem_e[b],
            )
            pltpu.async_copy(x_hbm.at[gidx[b]], xg[b], sem_x[b])

        def wait_scatter(b):
            pltpu.make_async_copy(xg[b], acc_sh.at[sidx[b]], sem_s[b]).wait()

        issue_gidx(0, 0)
        issue_sidx(0, 0)
        issue_gidx(1, 1)
        pltpu.make_async_copy(
            gi_hbm.at[pl.ds(c * E + lo + s * CHUNK, CHUNK)], gidx0, sem_gi0
        ).wait()
        issue_data(0, 0)

        @pl.loop(0, chunks_per_sub + 1, step=2)
        def _(t_outer):
          for b in (0, 1):
            bn = 1 - b
            t = t_outer + b
            if True:
                @pl.when((t >= 1) & (ci_of(t - 1) < n_chunks))
                def _():
                    wait_scatter(bn)

                @pl.when(valid(t + 1))
                def _():
                    issue_sidx(t + 1, bn)
                    pltpu.make_async_copy(
                        gi_hbm.at[
                            pl.ds(c * E + lo + ci_of(t + 1) * CHUNK, CHUNK)
                        ],
                        gidx[bn], sem_gi[bn],
                    ).wait()
                    issue_data(t + 1, bn)

                @pl.when(valid(t))
                def _():
                    pltpu.make_async_copy(
                        e_hbm.at[c, pl.ds(ci_of(t) * CHUNK, CHUNK)], ev[b],
                        sem_e[b],
                    ).wait()
                    pltpu.make_async_copy(
                        x_hbm.at[gidx[b]], xg[b], sem_x[b]
                    ).wait()

                    @pl.when(valid(t + 2))
                    def _():
                        issue_gidx(t + 2, b)

                    @pl.loop(0, CHUNK)
                    def _(i):
                        for j in range(H // LANES):
                            sl = pl.ds(j * LANES, LANES)
                            xg[b][i, sl] = jnp.maximum(
                                xg[b][i, sl] + ev[b][i, sl], 0.0
                            )

                    pltpu.make_async_copy(
                        gi_hbm.at[
                            pl.ds((1 - c) * E + ci_of(t) * CHUNK, CHUNK)
                        ],
                        sidx[b], sem_si[b],
                    ).wait()
                    pltpu.async_copy(
                        xg[b], acc_sh.at[sidx[b]], sem_s[b], add=True
                    )

        plsc.subcore_barrier()
        pltpu.sync_copy(
            acc_sh.at[pl.ds(base_rows, rows_per_sub)],
            out_hbm.at[c, pl.ds(base_rows, rows_per_sub)],
        )

        @pl.when(s == NSUB - 1)
        def _():
            pltpu.sync_copy(
                acc_sh.at[pl.ds(NSUB * rows_per_sub, extra_rows)],
                out_hbm.at[c, pl.ds(NSUB * rows_per_sub, extra_rows)],
            )

    return k(g_idx, x, e_stack)


def _node_mlp_body(x_ref, agg_ref, aggb_ref, w1_ref, b1_ref,
                   w2_ref, b2_ref, w3_ref, b3_ref, out_ref):
    w1 = w1_ref[...]
    b1 = b1_ref[...]
    w2 = w2_ref[...]
    b2 = b2_ref[...]

    def head(a):
        h = jnp.maximum(
            jnp.dot(a, w1, preferred_element_type=jnp.float32) + b1, 0.0
        )
        return jnp.dot(h, w2, preferred_element_type=jnp.float32) + b2

    yf = head(agg_ref[0] + aggb_ref[0])
    yb = head(agg_ref[1] + aggb_ref[1])
    xb = x_ref[...]
    out = (
        jnp.dot(xb, w3_ref[0:H], preferred_element_type=jnp.float32)
        + jnp.dot(yf, w3_ref[H:2 * H], preferred_element_type=jnp.float32)
        + jnp.dot(yb, w3_ref[2 * H:3 * H], preferred_element_type=jnp.float32)
        + b3_ref[...]
    )
    out_ref[...] = out


def _node_mlp(x, aggr, aggr_b, W1, b1, W2, b2, W3, b3,
              block_n=1000):
    N = x.shape[0]
    return pl.pallas_call(
        _node_mlp_body,
        grid=(N // block_n,),
        in_specs=[
            pl.BlockSpec((block_n, H), lambda i: (i, 0)),
            pl.BlockSpec((2, block_n, H), lambda i: (0, i, 0)),
            pl.BlockSpec((2, block_n, H), lambda i: (0, i, 0)),
            pl.BlockSpec((H, 2 * H), lambda i: (0, 0)),
            pl.BlockSpec((1, 2 * H), lambda i: (0, 0)),
            pl.BlockSpec((2 * H, H), lambda i: (0, 0)),
            pl.BlockSpec((1, H), lambda i: (0, 0)),
            pl.BlockSpec((3 * H, H), lambda i: (0, 0)),
            pl.BlockSpec((1, H), lambda i: (0, 0)),
        ],
        out_specs=pl.BlockSpec((block_n, H), lambda i: (i, 0)),
        out_shape=jax.ShapeDtypeStruct((N, H), jnp.float32),
        compiler_params=pltpu.CompilerParams(
            dimension_semantics=("parallel",)
        ),
    )(x, aggr, aggr_b, W1, b1.reshape(1, -1), W2,
      b2.reshape(1, -1), W3, b3.reshape(1, -1))


def kernel(x, edge_index, edge_attr, W_ef, b_ef, W_eb, b_eb, W1, b1, W2, b2,
           W3, b3):
    E = edge_index.shape[1]
    s1 = 3 * E // 8
    w_stack = jnp.concatenate([W_ef, W_eb], axis=1).astype(jnp.bfloat16)
    b_stack = jnp.concatenate([b_ef, b_eb]).reshape(1, 2 * H)
    g_idx = edge_index.reshape(-1)
    e_a = _edge_linear(edge_attr, w_stack, b_stack, 0, s1)
    e_b = _edge_linear(edge_attr, w_stack, b_stack, s1, E - s1)
    aggr_a = _sc_aggregate(g_idx, x, e_a, 0)
    aggr_b = _sc_aggregate(g_idx, x, e_b, s1)
    return _node_mlp(x, aggr_a, aggr_b, W1, b1, W2, b2, W3, b3)


# speedup vs baseline: 1.0177x; 1.0177x over previous
import dataclasses
import functools

import numpy as np

import jax
import jax.numpy as jnp
from jax import lax
from jax.experimental import pallas as pl
from jax.experimental.pallas import tpu as pltpu
from jax.experimental.pallas import tpu_sc as plsc

H = 128
CHUNK = 80
NSUB = 16
LANES = 16

def _edge_linear_body(ea_ref, w_ref, b_ref, out_ref):
    a = ea_ref[...].astype(jnp.bfloat16)
    y = jnp.dot(a, w_ref[...], preferred_element_type=jnp.float32) + b_ref[...]
    out_ref[0] = y[:, :H]
    out_ref[1] = y[:, H:]


def _edge_linear(edge_attr, w_stack, b_stack, lo, span, block_e=2000):
    return pl.pallas_call(
        _edge_linear_body,
        grid=(span // block_e,),
        in_specs=[
            pl.BlockSpec((block_e, H), lambda i: (i + lo // block_e, 0)),
            pl.BlockSpec((H, 2 * H), lambda i: (0, 0)),
            pl.BlockSpec((1, 2 * H), lambda i: (0, 0)),
        ],
        out_specs=pl.BlockSpec((2, block_e, H), lambda i: (0, i, 0)),
        out_shape=jax.ShapeDtypeStruct((2, span, H), jnp.float32),
        compiler_params=pltpu.CompilerParams(
            dimension_semantics=("parallel",)
        ),
    )(edge_attr, w_stack, b_stack)


def _sc_aggregate(g_idx, x, e_stack, lo):
    N = x.shape[0]
    E = g_idx.shape[0] // 2
    span = e_stack.shape[1]
    n_chunks = span // CHUNK
    chunks_per_sub = (n_chunks + NSUB - 1) // NSUB
    rows_per_sub = 624
    extra_rows = N - NSUB * rows_per_sub
    full_zero = rows_per_sub // CHUNK
    rem_zero = rows_per_sub % CHUNK

    mesh = plsc.VectorSubcoreMesh(core_axis_name="c", subcore_axis_name="s")

    @functools.partial(
        pl.kernel,
        out_type=jax.ShapeDtypeStruct((2, N, H), jnp.float32),
        mesh=mesh,
        scratch_types=[
            pltpu.VMEM((CHUNK,), jnp.int32),
            pltpu.VMEM((CHUNK,), jnp.int32),
            pltpu.VMEM((CHUNK,), jnp.int32),
            pltpu.VMEM((CHUNK,), jnp.int32),
            pltpu.VMEM((CHUNK, H), jnp.float32),
            pltpu.VMEM((CHUNK, H), jnp.float32),
            pltpu.VMEM((CHUNK, H), jnp.float32),
            pltpu.VMEM((CHUNK, H), jnp.float32),
            pltpu.VMEM_SHARED((N, H), jnp.float32),
        ] + [pltpu.SemaphoreType.DMA] * 10,
    )
    def k(gi_hbm, x_hbm, e_hbm, out_hbm, gidx0, gidx1, sidx0, sidx1, xg0,
          xg1, e0, e1, acc_sh, sem_gi0, sem_gi1, sem_si0, sem_si1, sem_x0,
          sem_x1, sem_e0, sem_e1, sem_s0, sem_s1):
        c = lax.axis_index("c")
        s = lax.axis_index("s")
        gidx = [gidx0, gidx1]
        sidx = [sidx0, sidx1]
        xg = [xg0, xg1]
        ev = [e0, e1]
        sem_gi = [sem_gi0, sem_gi1]
        sem_si = [sem_si0, sem_si1]
        sem_x = [sem_x0, sem_x1]
        sem_e = [sem_e0, sem_e1]
        sem_s = [sem_s0, sem_s1]

        @pl.loop(0, CHUNK)
        def _(i):
            for j in range(H // LANES):
                xg0[i, pl.ds(j * LANES, LANES)] = jnp.zeros(
                    (LANES,), jnp.float32
                )

        base_rows = s * rows_per_sub

        @pl.loop(0, full_zero)
        def _(t):
            pltpu.sync_copy(
                xg0, acc_sh.at[pl.ds(base_rows + t * CHUNK, CHUNK)]
            )

        pltpu.sync_copy(
            xg0.at[pl.ds(0, rem_zero)],
            acc_sh.at[pl.ds(base_rows + full_zero * CHUNK, rem_zero)],
        )

        @pl.when(s == NSUB - 1)
        def _():
            pltpu.sync_copy(
                xg0.at[pl.ds(0, extra_rows)],
                acc_sh.at[pl.ds(NSUB * rows_per_sub, extra_rows)],
            )

        plsc.subcore_barrier()

        def ci_of(t):
            return t * NSUB + s

        def valid(t):
            return ci_of(t) < n_chunks

        def issue_gidx(t, b):
            pltpu.async_copy(
                gi_hbm.at[pl.ds(c * E + lo + ci_of(t) * CHUNK, CHUNK)],
                gidx[b], sem_gi[b],
            )

        def issue_sidx(t, b):
            pltpu.async_copy(
                gi_hbm.at[pl.ds((1 - c) * E + lo + ci_of(t) * CHUNK, CHUNK)],
                sidx[b], sem_si[b],
            )

        def issue_data(t, b):
            pltpu.async_copy(
                e_hbm.at[c, pl.ds(ci_of(t) * CHUNK, CHUNK)], ev[b],
                sem_e[b],
            )
            pltpu.async_copy(x_hbm.at[gidx[b]], xg[b], sem_x[b])

        def wait_scatter(b):
            pltpu.make_async_copy(xg[b], acc_sh.at[sidx[b]], sem_s[b]).wait()

        issue_gidx(0, 0)
        issue_sidx(0, 0)
        issue_gidx(1, 1)
        pltpu.make_async_copy(
            gi_hbm.at[pl.ds(c * E + lo + s * CHUNK, CHUNK)], gidx0, sem_gi0
        ).wait()
        issue_data(0, 0)

        @pl.loop(0, chunks_per_sub + 1, step=2)
        def _(t_outer):
          for b in (0, 1):
            bn = 1 - b
            t = t_outer + b
            if True:
                @pl.when((t >= 1) & (ci_of(t - 1) < n_chunks))
                def _():
                    wait_scatter(bn)

                @pl.when(valid(t + 1))
                def _():
                    issue_sidx(t + 1, bn)
                    pltpu.make_async_copy(
                        gi_hbm.at[
                            pl.ds(c * E + lo + ci_of(t + 1) * CHUNK, CHUNK)
                        ],
                        gidx[bn], sem_gi[bn],
                    ).wait()
                    issue_data(t + 1, bn)

                @pl.when(valid(t))
                def _():
                    pltpu.make_async_copy(
                        e_hbm.at[c, pl.ds(ci_of(t) * CHUNK, CHUNK)], ev[b],
                        sem_e[b],
                    ).wait()
                    pltpu.make_async_copy(
                        x_hbm.at[gidx[b]], xg[b], sem_x[b]
                    ).wait()

                    @pl.when(valid(t + 2))
                    def _():
                        issue_gidx(t + 2, b)

                    @plsc.parallel_loop(0, CHUNK, unroll=4)
                    def _(i):
                        for j in range(H // LANES):
                            sl = pl.ds(j * LANES, LANES)
                            xg[b][i, sl] = jnp.maximum(
                                xg[b][i, sl] + ev[b][i, sl], 0.0
                            )

                    pltpu.make_async_copy(
                        gi_hbm.at[
                            pl.ds((1 - c) * E + ci_of(t) * CHUNK, CHUNK)
                        ],
                        sidx[b], sem_si[b],
                    ).wait()
                    pltpu.async_copy(
                        xg[b], acc_sh.at[sidx[b]], sem_s[b], add=True
                    )

        plsc.subcore_barrier()
        pltpu.sync_copy(
            acc_sh.at[pl.ds(base_rows, rows_per_sub)],
            out_hbm.at[c, pl.ds(base_rows, rows_per_sub)],
        )

        @pl.when(s == NSUB - 1)
        def _():
            pltpu.sync_copy(
                acc_sh.at[pl.ds(NSUB * rows_per_sub, extra_rows)],
                out_hbm.at[c, pl.ds(NSUB * rows_per_sub, extra_rows)],
            )

    return k(g_idx, x, e_stack)


def _node_mlp_body(x_ref, agg_ref, aggb_ref, w1_ref, b1_ref,
                   w2_ref, b2_ref, w3_ref, b3_ref, out_ref):
    w1 = w1_ref[...]
    b1 = b1_ref[...]
    w2 = w2_ref[...]
    b2 = b2_ref[...]

    def head(a):
        h = jnp.maximum(
            jnp.dot(a, w1, preferred_element_type=jnp.float32) + b1, 0.0
        )
        return jnp.dot(h, w2, preferred_element_type=jnp.float32) + b2

    yf = head(agg_ref[0] + aggb_ref[0])
    yb = head(agg_ref[1] + aggb_ref[1])
    xb = x_ref[...]
    out = (
        jnp.dot(xb, w3_ref[0:H], preferred_element_type=jnp.float32)
        + jnp.dot(yf, w3_ref[H:2 * H], preferred_element_type=jnp.float32)
        + jnp.dot(yb, w3_ref[2 * H:3 * H], preferred_element_type=jnp.float32)
        + b3_ref[...]
    )
    out_ref[...] = out


def _node_mlp(x, aggr, aggr_b, W1, b1, W2, b2, W3, b3,
              block_n=1000):
    N = x.shape[0]
    return pl.pallas_call(
        _node_mlp_body,
        grid=(N // block_n,),
        in_specs=[
            pl.BlockSpec((block_n, H), lambda i: (i, 0)),
            pl.BlockSpec((2, block_n, H), lambda i: (0, i, 0)),
            pl.BlockSpec((2, block_n, H), lambda i: (0, i, 0)),
            pl.BlockSpec((H, 2 * H), lambda i: (0, 0)),
            pl.BlockSpec((1, 2 * H), lambda i: (0, 0)),
            pl.BlockSpec((2 * H, H), lambda i: (0, 0)),
            pl.BlockSpec((1, H), lambda i: (0, 0)),
            pl.BlockSpec((3 * H, H), lambda i: (0, 0)),
            pl.BlockSpec((1, H), lambda i: (0, 0)),
        ],
        out_specs=pl.BlockSpec((block_n, H), lambda i: (i, 0)),
        out_shape=jax.ShapeDtypeStruct((N, H), jnp.float32),
        compiler_params=pltpu.CompilerParams(
            dimension_semantics=("parallel",)
        ),
    )(x, aggr, aggr_b, W1, b1.reshape(1, -1), W2,
      b2.reshape(1, -1), W3, b3.reshape(1, -1))


def kernel(x, edge_index, edge_attr, W_ef, b_ef, W_eb, b_eb, W1, b1, W2, b2,
           W3, b3):
    E = edge_index.shape[1]
    s1 = E // 2
    w_stack = jnp.concatenate([W_ef, W_eb], axis=1).astype(jnp.bfloat16)
    b_stack = jnp.concatenate([b_ef, b_eb]).reshape(1, 2 * H)
    g_idx = edge_index.reshape(-1)
    e_a = _edge_linear(edge_attr, w_stack, b_stack, 0, s1)
    e_b = _edge_linear(edge_attr, w_stack, b_stack, s1, E - s1)
    aggr_a = _sc_aggregate(g_idx, x, e_a, 0)
    aggr_b = _sc_aggregate(g_idx, x, e_b, s1)
    return _node_mlp(x, aggr_a, aggr_b, W1, b1, W2, b2, W3, b3)


# speedup vs baseline: 1.0355x; 1.0175x over previous
import dataclasses
import functools

import numpy as np

import jax
import jax.numpy as jnp
from jax import lax
from jax.experimental import pallas as pl
from jax.experimental.pallas import tpu as pltpu
from jax.experimental.pallas import tpu_sc as plsc

H = 128
CHUNK = 80
NSUB = 16
LANES = 16

def _edge_linear_body(ea_ref, w_ref, b_ref, out_ref):
    a = ea_ref[...].astype(jnp.bfloat16)
    y = jnp.dot(a, w_ref[...], preferred_element_type=jnp.float32) + b_ref[...]
    out_ref[0] = y[:, :H]
    out_ref[1] = y[:, H:]


def _edge_linear(edge_attr, w_stack, b_stack, lo, span, block_e=1280):
    return pl.pallas_call(
        _edge_linear_body,
        grid=(span // block_e,),
        in_specs=[
            pl.BlockSpec((block_e, H), lambda i: (i + lo // block_e, 0)),
            pl.BlockSpec((H, 2 * H), lambda i: (0, 0)),
            pl.BlockSpec((1, 2 * H), lambda i: (0, 0)),
        ],
        out_specs=pl.BlockSpec((2, block_e, H), lambda i: (0, i, 0)),
        out_shape=jax.ShapeDtypeStruct((2, span, H), jnp.float32),
        compiler_params=pltpu.CompilerParams(
            dimension_semantics=("parallel",)
        ),
    )(edge_attr, w_stack, b_stack)


def _sc_aggregate(g_idx, x, e_stack, lo):
    N = x.shape[0]
    E = g_idx.shape[0] // 2
    span = e_stack.shape[1]
    n_chunks = span // CHUNK
    assert span % (NSUB * CHUNK) == 0, span
    chunks_per_sub = n_chunks // NSUB
    rows_per_sub = 624
    extra_rows = N - NSUB * rows_per_sub
    full_zero = rows_per_sub // CHUNK
    rem_zero = rows_per_sub % CHUNK

    mesh = plsc.VectorSubcoreMesh(core_axis_name="c", subcore_axis_name="s")

    @functools.partial(
        pl.kernel,
        out_type=jax.ShapeDtypeStruct((2, N, H), jnp.float32),
        mesh=mesh,
        scratch_types=[
            pltpu.VMEM((CHUNK,), jnp.int32),
            pltpu.VMEM((CHUNK,), jnp.int32),
            pltpu.VMEM((CHUNK, H), jnp.float32),
            pltpu.VMEM((CHUNK, H), jnp.float32),
            pltpu.VMEM((CHUNK, H), jnp.float32),
            pltpu.VMEM((CHUNK, H), jnp.float32),
            pltpu.VMEM((chunks_per_sub * CHUNK,), jnp.int32),
            pltpu.VMEM_SHARED((N, H), jnp.float32),
        ] + [pltpu.SemaphoreType.DMA] * 8,
    )
    def k(gi_hbm, x_hbm, e_hbm, out_hbm, sidx0, sidx1, xg0, xg1, e0, e1,
          gbulk, acc_sh, sem_si0, sem_si1, sem_x0, sem_x1, sem_e0, sem_e1,
          sem_s0, sem_s1):
        c = lax.axis_index("c")
        s = lax.axis_index("s")
        sidx = [sidx0, sidx1]
        xg = [xg0, xg1]
        ev = [e0, e1]
        sem_si = [sem_si0, sem_si1]
        sem_x = [sem_x0, sem_x1]
        sem_e = [sem_e0, sem_e1]
        sem_s = [sem_s0, sem_s1]

        edges_per_sub = chunks_per_sub * CHUNK
        gcp = pltpu.async_copy(
            gi_hbm.at[pl.ds(c * E + lo + s * edges_per_sub, edges_per_sub)],
            gbulk, sem_s0,
        )

        @pl.loop(0, CHUNK)
        def _(i):
            for j in range(H // LANES):
                xg0[i, pl.ds(j * LANES, LANES)] = jnp.zeros(
                    (LANES,), jnp.float32
                )

        base_rows = s * rows_per_sub

        @pl.loop(0, full_zero)
        def _(t):
            pltpu.sync_copy(
                xg0, acc_sh.at[pl.ds(base_rows + t * CHUNK, CHUNK)]
            )

        pltpu.sync_copy(
            xg0.at[pl.ds(0, rem_zero)],
            acc_sh.at[pl.ds(base_rows + full_zero * CHUNK, rem_zero)],
        )

        @pl.when(s == NSUB - 1)
        def _():
            pltpu.sync_copy(
                xg0.at[pl.ds(0, extra_rows)],
                acc_sh.at[pl.ds(NSUB * rows_per_sub, extra_rows)],
            )

        gcp.wait()
        plsc.subcore_barrier()

        def ci_of(t):
            return s * chunks_per_sub + t

        def issue_sidx(t, b):
            pltpu.async_copy(
                gi_hbm.at[pl.ds((1 - c) * E + lo + ci_of(t) * CHUNK, CHUNK)],
                sidx[b], sem_si[b],
            )

        def issue_data(t, b):
            pltpu.async_copy(
                e_hbm.at[c, pl.ds(ci_of(t) * CHUNK, CHUNK)], ev[b],
                sem_e[b],
            )
            pltpu.async_copy(
                x_hbm.at[gbulk.at[pl.ds(t * CHUNK, CHUNK)]], xg[b], sem_x[b]
            )

        def wait_data(t, b):
            pltpu.make_async_copy(
                e_hbm.at[c, pl.ds(ci_of(t) * CHUNK, CHUNK)], ev[b],
                sem_e[b],
            ).wait()
            pltpu.make_async_copy(
                x_hbm.at[gbulk.at[pl.ds(t * CHUNK, CHUNK)]], xg[b], sem_x[b]
            ).wait()

        def wait_sidx(t, b):
            pltpu.make_async_copy(
                gi_hbm.at[pl.ds((1 - c) * E + lo + ci_of(t) * CHUNK, CHUNK)],
                sidx[b], sem_si[b],
            ).wait()

        def wait_scatter(b):
            pltpu.make_async_copy(xg[b], acc_sh.at[sidx[b]], sem_s[b]).wait()

        issue_sidx(0, 0)
        issue_data(0, 0)

        @pl.loop(0, chunks_per_sub + 1, step=2)
        def _(t_outer):
          for b in (0, 1):
            bn = 1 - b
            t = t_outer + b
            if True:
                @pl.when((t >= 1) & (t - 1 < chunks_per_sub))
                def _():
                    wait_scatter(bn)

                @pl.when(t + 1 < chunks_per_sub)
                def _():
                    issue_sidx(t + 1, bn)
                    issue_data(t + 1, bn)

                @pl.when(t < chunks_per_sub)
                def _():
                    wait_data(t, b)

                    @plsc.parallel_loop(0, CHUNK, unroll=4)
                    def _(i):
                        for j in range(H // LANES):
                            sl = pl.ds(j * LANES, LANES)
                            xg[b][i, sl] = jnp.maximum(
                                xg[b][i, sl] + ev[b][i, sl], 0.0
                            )

                    wait_sidx(t, b)
                    pltpu.async_copy(
                        xg[b], acc_sh.at[sidx[b]], sem_s[b], add=True
                    )

        plsc.subcore_barrier()
        pltpu.sync_copy(
            acc_sh.at[pl.ds(base_rows, rows_per_sub)],
            out_hbm.at[c, pl.ds(base_rows, rows_per_sub)],
        )

        @pl.when(s == NSUB - 1)
        def _():
            pltpu.sync_copy(
                acc_sh.at[pl.ds(NSUB * rows_per_sub, extra_rows)],
                out_hbm.at[c, pl.ds(NSUB * rows_per_sub, extra_rows)],
            )

    return k(g_idx, x, e_stack)


def _node_mlp_body(x_ref, agg_ref, aggb_ref, aggc_ref, w1_ref, b1_ref,
                   w2_ref, b2_ref, w3_ref, b3_ref, out_ref):
    w1 = w1_ref[...]
    b1 = b1_ref[...]
    w2 = w2_ref[...]
    b2 = b2_ref[...]

    def head(a):
        h = jnp.maximum(
            jnp.dot(a, w1, preferred_element_type=jnp.float32) + b1, 0.0
        )
        return jnp.dot(h, w2, preferred_element_type=jnp.float32) + b2

    yf = head(agg_ref[0] + aggb_ref[0] + aggc_ref[0])
    yb = head(agg_ref[1] + aggb_ref[1] + aggc_ref[1])
    xb = x_ref[...]
    out = (
        jnp.dot(xb, w3_ref[0:H], preferred_element_type=jnp.float32)
        + jnp.dot(yf, w3_ref[H:2 * H], preferred_element_type=jnp.float32)
        + jnp.dot(yb, w3_ref[2 * H:3 * H], preferred_element_type=jnp.float32)
        + b3_ref[...]
    )
    out_ref[...] = out


def _node_mlp(x, aggr, aggr_b, aggr_c, W1, b1, W2, b2, W3, b3,
              block_n=1000):
    N = x.shape[0]
    return pl.pallas_call(
        _node_mlp_body,
        grid=(N // block_n,),
        in_specs=[
            pl.BlockSpec((block_n, H), lambda i: (i, 0)),
            pl.BlockSpec((2, block_n, H), lambda i: (0, i, 0)),
            pl.BlockSpec((2, block_n, H), lambda i: (0, i, 0)),
            pl.BlockSpec((2, block_n, H), lambda i: (0, i, 0)),
            pl.BlockSpec((H, 2 * H), lambda i: (0, 0)),
            pl.BlockSpec((1, 2 * H), lambda i: (0, 0)),
            pl.BlockSpec((2 * H, H), lambda i: (0, 0)),
            pl.BlockSpec((1, H), lambda i: (0, 0)),
            pl.BlockSpec((3 * H, H), lambda i: (0, 0)),
            pl.BlockSpec((1, H), lambda i: (0, 0)),
        ],
        out_specs=pl.BlockSpec((block_n, H), lambda i: (i, 0)),
        out_shape=jax.ShapeDtypeStruct((N, H), jnp.float32),
        compiler_params=pltpu.CompilerParams(
            dimension_semantics=("parallel",)
        ),
    )(x, aggr, aggr_b, aggr_c, W1, b1.reshape(1, -1), W2,
      b2.reshape(1, -1), W3, b3.reshape(1, -1))


def kernel(x, edge_index, edge_attr, W_ef, b_ef, W_eb, b_eb, W1, b1, W2, b2,
           W3, b3):
    E = edge_index.shape[1]
    unit = NSUB * CHUNK
    u = E // unit
    s1 = (u // 3 + 1) * unit
    s2 = s1 + (u // 3) * unit
    w_stack = jnp.concatenate([W_ef, W_eb], axis=1).astype(jnp.bfloat16)
    b_stack = jnp.concatenate([b_ef, b_eb]).reshape(1, 2 * H)
    g_idx = edge_index.reshape(-1)
    e_a = _edge_linear(edge_attr, w_stack, b_stack, 0, s1)
    e_b = _edge_linear(edge_attr, w_stack, b_stack, s1, s2 - s1)
    e_c = _edge_linear(edge_attr, w_stack, b_stack, s2, E - s2)
    aggr_a = _sc_aggregate(g_idx, x, e_a, 0)
    aggr_b = _sc_aggregate(g_idx, x, e_b, s1)
    aggr_c = _sc_aggregate(g_idx, x, e_c, s2)
    return _node_mlp(x, aggr_a, aggr_b, aggr_c, W1, b1, W2, b2, W3, b3)
